# baseline (speedup 1.0000x reference)
import functools

import jax
import jax.numpy as jnp
from jax import lax
from jax.experimental import pallas as pl
from jax.experimental.pallas import tpu as pltpu


def _conv(x, w, stride):
    return lax.conv_general_dilated(
        x, w, (stride, stride), 'SAME',
        dimension_numbers=('NCHW', 'OIHW', 'NCHW'))


def _up_conv(x, w, precision=None):
    w0, w1, w2 = w[:, :, 0], w[:, :, 1], w[:, :, 2]
    kr = jnp.stack([w0, w0 + w1, w1 + w2, w2], axis=2)
    c0, c1, c2 = kr[..., 0], kr[..., 1], kr[..., 2]
    k4 = jnp.stack([c0, c0 + c1, c1 + c2, c2], axis=3)
    return lax.conv_general_dilated(
        x, k4, (1, 1), ((2, 2), (2, 2)), lhs_dilation=(2, 2),
        dimension_numbers=('NCHW', 'OIHW', 'NCHW'),
        precision=precision)


def _avgpool(x, win=11):
    s = lax.reduce_window(x, 0.0, lax.add, (1, 1, win, 1), (1, 1, 1, 1), 'VALID')
    s = lax.reduce_window(s, 0.0, lax.add, (1, 1, 1, win), (1, 1, 1, 1), 'VALID')
    return s / float(win * win)


def _ssim(x, y):
    C1 = 0.01 ** 2
    C2 = 0.03 ** 2
    mx = _avgpool(x)
    my = _avgpool(y)
    sx = _avgpool(x * x) - mx * mx
    sy = _avgpool(y * y) - my * my
    sxy = _avgpool(x * y) - mx * my
    num = (2.0 * mx * my + C1) * (2.0 * sxy + C2)
    den = (mx * mx + my * my + C1) * (sx + sy + C2)
    return jnp.mean(num / den)


_KB = 1024


def _pq_kernel(invt_ref, zg_ref, cb_ref,
               logits_ref, codes_ref, ent_ref,
               m_scr, z_scr, s_scr, bv_scr, bi_scr):
    kk = pl.program_id(1)
    nk = pl.num_programs(1)

    zg = zg_ref[0]
    cb = cb_ref[0]
    invt = invt_ref[0, 0]

    @pl.when(kk == 0)
    def _init():
        m_scr[...] = jnp.full_like(m_scr[...], -jnp.inf)
        z_scr[...] = jnp.zeros_like(z_scr[...])
        s_scr[...] = jnp.zeros_like(s_scr[...])
        bv_scr[...] = jnp.full_like(bv_scr[...], -jnp.inf)
        bi_scr[...] = jnp.zeros_like(bi_scr[...])

    dots = lax.dot_general(zg, cb, (((1,), (1,)), ((), ())),
                           preferred_element_type=jnp.float32)
    z2 = jnp.sum(zg * zg, axis=1, keepdims=True)
    c2 = jnp.sum(cb * cb, axis=1)[None, :]
    logits = 2.0 * dots - z2 - c2
    logits_ref[0] = logits

    bmax = jnp.max(logits, axis=1, keepdims=True)
    iota = lax.broadcasted_iota(jnp.int32, logits.shape, 1)
    local = jnp.min(jnp.where(logits == bmax, iota, jnp.int32(2 ** 30)),
                    axis=1, keepdims=True)

    upd = bmax > bv_scr[...]
    bv_scr[...] = jnp.where(upd, bmax, bv_scr[...])
    bi_scr[...] = jnp.where(upd, local + kk * _KB, bi_scr[...])

    l = logits * invt
    lmax = bmax * invt
    m_old = m_scr[...]
    m_new = jnp.maximum(m_old, lmax)
    alpha = jnp.exp(m_old - m_new)
    e = jnp.exp(l - m_new)
    z_scr[...] = z_scr[...] * alpha + jnp.sum(e, axis=1, keepdims=True)
    s_scr[...] = s_scr[...] * alpha + jnp.sum(e * l, axis=1, keepdims=True)
    m_scr[...] = m_new

    @pl.when(kk == nk - 1)
    def _fin():
        m = m_scr[...]
        z = z_scr[...]
        s = s_scr[...]
        ent_ref[0] = (m + jnp.log(z)) - s / z
        codes_ref[0] = bi_scr[...]


def _pq_quantize(zg, codebook, invt):
    G, T, d = zg.shape
    m, k, _ = codebook.shape
    nk = k // _KB
    grid = (G, nk)
    return pl.pallas_call(
        _pq_kernel,
        grid=grid,
        in_specs=[
            pl.BlockSpec(memory_space=pltpu.SMEM),
            pl.BlockSpec((1, T, d), lambda i, j: (i, 0, 0)),
            pl.BlockSpec((1, _KB, d), lambda i, j: (i % m, j, 0)),
        ],
        out_specs=[
            pl.BlockSpec((1, T, _KB), lambda i, j: (i, 0, j)),
            pl.BlockSpec((1, T, 1), lambda i, j: (i, 0, 0)),
            pl.BlockSpec((1, T, 1), lambda i, j: (i, 0, 0)),
        ],
        out_shape=[
            jax.ShapeDtypeStruct((G, T, k), jnp.float32),
            jax.ShapeDtypeStruct((G, T, 1), jnp.int32),
            jax.ShapeDtypeStruct((G, T, 1), jnp.float32),
        ],
        scratch_shapes=[
            pltpu.VMEM((T, 1), jnp.float32),
            pltpu.VMEM((T, 1), jnp.float32),
            pltpu.VMEM((T, 1), jnp.float32),
            pltpu.VMEM((T, 1), jnp.float32),
            pltpu.VMEM((T, 1), jnp.int32),
        ],
        compiler_params=pltpu.CompilerParams(
            dimension_semantics=("arbitrary", "arbitrary")),
    )(invt, zg, codebook)


def _sc_gather(table, idx):
    from jax.experimental.pallas import tpu_sc as plsc
    info = plsc.get_sparse_core_info()
    nc, ns = info.num_cores, info.num_subcores
    nw = nc * ns
    n, d = idx.shape[0], table.shape[1]
    b_per_w = n // nw
    mesh = plsc.VectorSubcoreMesh(core_axis_name="c", subcore_axis_name="s")

    @functools.partial(
        pl.kernel, mesh=mesh,
        out_type=jax.ShapeDtypeStruct((n, d), jnp.float32),
        scratch_types=[
            pltpu.VMEM((b_per_w,), jnp.int32),
            pltpu.VMEM((b_per_w, d), jnp.float32),
            pltpu.SemaphoreType.DMA,
        ],
    )
    def gather(table_hbm, idx_hbm, out_hbm, idx_v, rows_v, sem):
        wid = lax.axis_index("s") * nc + lax.axis_index("c")
        base = wid * b_per_w
        pltpu.sync_copy(idx_hbm.at[pl.ds(base, b_per_w)], idx_v)
        pltpu.async_copy(table_hbm.at[idx_v], rows_v, sem).wait()
        pltpu.sync_copy(rows_v, out_hbm.at[pl.ds(base, b_per_w)])

    return gather(table, idx)


def kernel(image, temp, enc_w1, enc_w2, enc_w3, enc_w4,
           dec_w1, dec_w2, dec_w3, dec_w4, codebook):
    h = jax.nn.relu(_conv(image, enc_w1, 2))
    h = jax.nn.relu(_conv(h, enc_w2, 2))
    h = jax.nn.relu(_conv(h, enc_w3, 2))
    z = _conv(h, enc_w4, 2)
    B, C, hh, ww = z.shape
    m, k, d = codebook.shape
    T = hh * ww

    zg = z.reshape(B, m, d, T).transpose(0, 1, 3, 2)
    invt = (1.0 / jnp.asarray(temp, jnp.float32)).reshape(1, 1)

    logits_g, codes_g, ent_g = _pq_quantize(
        zg.reshape(B * m, T, d), codebook, invt)

    logits = logits_g.reshape(B, m, T, k)
    trueCodes = codes_g.reshape(B, m, T)
    reg = jnp.mean(ent_g)

    gidx = (trueCodes + (jnp.arange(m, dtype=jnp.int32) * k)[None, :, None])
    n = B * m * T
    npad = ((n + 255) // 256) * 256
    gidx = jnp.pad(gidx.reshape(n), (0, npad - n))
    table = jnp.pad(codebook.reshape(m * k, d), ((0, 0), (0, 128 - d)))
    qhard = _sc_gather(table, gidx)[:n, :d].reshape(B, m, T, d)
    quantized = qhard.transpose(0, 1, 3, 2).reshape(B, C, hh, ww)

    h = jax.nn.relu(_up_conv(quantized, dec_w1, lax.Precision.HIGHEST))
    h = jax.nn.relu(_up_conv(h, dec_w2, lax.Precision.HIGHEST))
    h = jax.nn.relu(_up_conv(h, dec_w3))
    restored = _up_conv(h, dec_w4)

    ssimLoss = 1.0 - _ssim(image, restored)
    diff = restored - image
    l1l2Loss = jnp.mean(jnp.abs(diff)) + jnp.mean(diff * diff)

    return ((ssimLoss, l1l2Loss, reg), (restored, trueCodes, quantized, logits))


# speedup vs baseline: 1.0246x; 1.0246x over previous
import functools

import jax
import jax.numpy as jnp
from jax import lax
from jax.experimental import pallas as pl
from jax.experimental.pallas import tpu as pltpu


def _conv(x, w, stride):
    return lax.conv_general_dilated(
        x, w, (stride, stride), 'SAME',
        dimension_numbers=('NCHW', 'OIHW', 'NCHW'))


def _up_conv(x, w, precision=None):
    w0, w1, w2 = w[:, :, 0], w[:, :, 1], w[:, :, 2]
    kr = jnp.stack([w0, w0 + w1, w1 + w2, w2], axis=2)
    c0, c1, c2 = kr[..., 0], kr[..., 1], kr[..., 2]
    k4 = jnp.stack([c0, c0 + c1, c1 + c2, c2], axis=3)
    return lax.conv_general_dilated(
        x, k4, (1, 1), ((2, 2), (2, 2)), lhs_dilation=(2, 2),
        dimension_numbers=('NCHW', 'OIHW', 'NCHW'),
        precision=precision)


def _avgpool(x, win=11):
    s = lax.reduce_window(x, 0.0, lax.add, (1, 1, win, 1), (1, 1, 1, 1), 'VALID')
    s = lax.reduce_window(s, 0.0, lax.add, (1, 1, 1, win), (1, 1, 1, 1), 'VALID')
    return s / float(win * win)


def _ssim(x, y):
    C1 = 0.01 ** 2
    C2 = 0.03 ** 2
    mx = _avgpool(x)
    my = _avgpool(y)
    sx = _avgpool(x * x) - mx * mx
    sy = _avgpool(y * y) - my * my
    sxy = _avgpool(x * y) - mx * my
    num = (2.0 * mx * my + C1) * (2.0 * sxy + C2)
    den = (mx * mx + my * my + C1) * (sx + sy + C2)
    return jnp.mean(num / den)


_KB = 2048


def _pq_kernel(invt_ref, zg_ref, cb_ref,
               logits_ref, codes_ref, ent_ref,
               m_scr, z_scr, s_scr, bv_scr, bi_scr):
    kk = pl.program_id(1)
    nk = pl.num_programs(1)

    zg = zg_ref[0]
    cb = cb_ref[0]
    invt = invt_ref[0, 0]

    @pl.when(kk == 0)
    def _init():
        m_scr[...] = jnp.full_like(m_scr[...], -jnp.inf)
        z_scr[...] = jnp.zeros_like(z_scr[...])
        s_scr[...] = jnp.zeros_like(s_scr[...])
        bv_scr[...] = jnp.full_like(bv_scr[...], -jnp.inf)
        bi_scr[...] = jnp.zeros_like(bi_scr[...])

    dots = lax.dot_general(zg, cb, (((1,), (1,)), ((), ())),
                           preferred_element_type=jnp.float32)
    z2 = jnp.sum(zg * zg, axis=1, keepdims=True)
    c2 = jnp.sum(cb * cb, axis=1)[None, :]
    logits = 2.0 * dots - z2 - c2
    logits_ref[0] = logits

    bmax = jnp.max(logits, axis=1, keepdims=True)
    iota = lax.broadcasted_iota(jnp.int32, logits.shape, 1)
    local = jnp.min(jnp.where(logits == bmax, iota, jnp.int32(2 ** 30)),
                    axis=1, keepdims=True)

    upd = bmax > bv_scr[...]
    bv_scr[...] = jnp.where(upd, bmax, bv_scr[...])
    bi_scr[...] = jnp.where(upd, local + kk * _KB, bi_scr[...])

    l = logits * invt
    lmax = bmax * invt
    m_old = m_scr[...]
    m_new = jnp.maximum(m_old, lmax)
    alpha = jnp.exp(m_old - m_new)
    e = jnp.exp(l - m_new)
    z_scr[...] = z_scr[...] * alpha + jnp.sum(e, axis=1, keepdims=True)
    s_scr[...] = s_scr[...] * alpha + jnp.sum(e * l, axis=1, keepdims=True)
    m_scr[...] = m_new

    @pl.when(kk == nk - 1)
    def _fin():
        m = m_scr[...]
        z = z_scr[...]
        s = s_scr[...]
        ent_ref[0] = (m + jnp.log(z)) - s / z
        codes_ref[0] = bi_scr[...]


def _pq_quantize(zg, codebook, invt):
    G, T, d = zg.shape
    m, k, _ = codebook.shape
    nk = k // _KB
    grid = (G, nk)
    return pl.pallas_call(
        _pq_kernel,
        grid=grid,
        in_specs=[
            pl.BlockSpec(memory_space=pltpu.SMEM),
            pl.BlockSpec((1, T, d), lambda i, j: (i, 0, 0)),
            pl.BlockSpec((1, _KB, d), lambda i, j: (i % m, j, 0)),
        ],
        out_specs=[
            pl.BlockSpec((1, T, _KB), lambda i, j: (i, 0, j)),
            pl.BlockSpec((1, T, 1), lambda i, j: (i, 0, 0)),
            pl.BlockSpec((1, T, 1), lambda i, j: (i, 0, 0)),
        ],
        out_shape=[
            jax.ShapeDtypeStruct((G, T, k), jnp.float32),
            jax.ShapeDtypeStruct((G, T, 1), jnp.int32),
            jax.ShapeDtypeStruct((G, T, 1), jnp.float32),
        ],
        scratch_shapes=[
            pltpu.VMEM((T, 1), jnp.float32),
            pltpu.VMEM((T, 1), jnp.float32),
            pltpu.VMEM((T, 1), jnp.float32),
            pltpu.VMEM((T, 1), jnp.float32),
            pltpu.VMEM((T, 1), jnp.int32),
        ],
        compiler_params=pltpu.CompilerParams(
            dimension_semantics=("arbitrary", "arbitrary")),
    )(invt, zg, codebook)


def _sc_gather(table, idx):
    from jax.experimental.pallas import tpu_sc as plsc
    info = plsc.get_sparse_core_info()
    nc, ns = info.num_cores, info.num_subcores
    nw = nc * ns
    n, d = idx.shape[0], table.shape[1]
    b_per_w = n // nw
    mesh = plsc.VectorSubcoreMesh(core_axis_name="c", subcore_axis_name="s")

    @functools.partial(
        pl.kernel, mesh=mesh,
        out_type=jax.ShapeDtypeStruct((n, d), jnp.float32),
        scratch_types=[
            pltpu.VMEM((b_per_w,), jnp.int32),
            pltpu.VMEM((b_per_w, d), jnp.float32),
            pltpu.SemaphoreType.DMA,
        ],
    )
    def gather(table_hbm, idx_hbm, out_hbm, idx_v, rows_v, sem):
        wid = lax.axis_index("s") * nc + lax.axis_index("c")
        base = wid * b_per_w
        pltpu.sync_copy(idx_hbm.at[pl.ds(base, b_per_w)], idx_v)
        pltpu.async_copy(table_hbm.at[idx_v], rows_v, sem).wait()
        pltpu.sync_copy(rows_v, out_hbm.at[pl.ds(base, b_per_w)])

    return gather(table, idx)


def kernel(image, temp, enc_w1, enc_w2, enc_w3, enc_w4,
           dec_w1, dec_w2, dec_w3, dec_w4, codebook):
    h = jax.nn.relu(_conv(image, enc_w1, 2))
    h = jax.nn.relu(_conv(h, enc_w2, 2))
    h = jax.nn.relu(_conv(h, enc_w3, 2))
    z = _conv(h, enc_w4, 2)
    B, C, hh, ww = z.shape
    m, k, d = codebook.shape
    T = hh * ww

    zg = z.reshape(B, m, d, T).transpose(0, 1, 3, 2)
    invt = (1.0 / jnp.asarray(temp, jnp.float32)).reshape(1, 1)

    logits_g, codes_g, ent_g = _pq_quantize(
        zg.reshape(B * m, T, d), codebook, invt)

    logits = logits_g.reshape(B, m, T, k)
    trueCodes = codes_g.reshape(B, m, T)
    reg = jnp.mean(ent_g)

    gidx = (trueCodes + (jnp.arange(m, dtype=jnp.int32) * k)[None, :, None])
    n = B * m * T
    npad = ((n + 255) // 256) * 256
    gidx = jnp.pad(gidx.reshape(n), (0, npad - n))
    table = codebook.reshape(m * k // 2, 2 * d)
    pairs = _sc_gather(table, gidx >> 1)[:n]
    qhard = jnp.where((gidx[:n] & 1)[:, None] == 0, pairs[:, :d], pairs[:, d:])
    qhard = qhard.reshape(B, m, T, d)
    quantized = qhard.transpose(0, 1, 3, 2).reshape(B, C, hh, ww)

    h = jax.nn.relu(_up_conv(quantized, dec_w1, lax.Precision.HIGHEST))
    h = jax.nn.relu(_up_conv(h, dec_w2, lax.Precision.HIGHEST))
    h = jax.nn.relu(_up_conv(h, dec_w3))
    restored = _up_conv(h, dec_w4)

    ssimLoss = 1.0 - _ssim(image, restored)
    diff = restored - image
    l1l2Loss = jnp.mean(jnp.abs(diff)) + jnp.mean(diff * diff)

    return ((ssimLoss, l1l2Loss, reg), (restored, trueCodes, quantized, logits))
